# Initial kernel scaffold; baseline (speedup 1.0000x reference)
import jax
import jax.numpy as jnp
from jax.experimental import pallas as pl


def kernel(x, edge_indexes, edge_attrs, W1, b1, W2, b2, Wc0, bc0, Wc1, bc1, Wconv, bconv, Wf, bf, att):
    raise NotImplementedError("write your pallas kernel here")


# speedup vs baseline: 7.8359x; 7.8359x over previous
import jax
import jax.numpy as jnp
from jax import lax
from jax.experimental import pallas as pl
from jax.experimental.pallas import tpu as pltpu
from jax.experimental.pallas import tpu_sc as plsc

N = 10000
NP = 10240
E = 320000
DH = 128
DOUT = 64

NC = 2
NS = 16
NW = NC * NS

CB = 128
EP = 327680
EPP = EP + 2 * CB
EROWS = EP // CB
ROWS_PER_TILE = EROWS // NW
NPT = NP // NS
NRI = NPT // CB

_mesh = plsc.VectorSubcoreMesh(
    core_axis_name="c", subcore_axis_name="s", num_cores=NC, num_subcores=NS)


def _deg_body(d0, d1, d2, one128, zer128, ridxf, out, didx0, didx1,
              ones_v, zbuf, rowidx, sem0, sem1, acc):
  core = lax.axis_index("c")
  sub = lax.axis_index("s")
  wid = core * NS + sub
  pltpu.sync_copy(one128, ones_v)
  ebase = wid * ROWS_PER_TILE * CB
  for c, d in enumerate((d0, d1, d2)):
    pltpu.sync_copy(zer128, zbuf)
    for j in range(NRI):
      pltpu.sync_copy(ridxf.at[pl.ds((sub * NRI + j) * CB, CB)], rowidx)
      pltpu.sync_copy(zbuf, acc.at[rowidx])
    plsc.subcore_barrier()
    pltpu.async_copy(d.at[pl.ds(ebase, CB)], didx0, sem0)
    pltpu.async_copy(d.at[pl.ds(ebase + CB, CB)], didx1, sem1)

    def grp(gi, carry, d=d):
      pltpu.make_async_copy(d.at[pl.ds(0, CB)], didx0, sem0).wait()
      pltpu.sync_copy(ones_v, acc.at[didx0], add=True)
      pltpu.async_copy(d.at[pl.ds(ebase + (2 * gi + 2) * CB, CB)], didx0,
                       sem0)
      pltpu.make_async_copy(d.at[pl.ds(0, CB)], didx1, sem1).wait()
      pltpu.sync_copy(ones_v, acc.at[didx1], add=True)
      pltpu.async_copy(d.at[pl.ds(ebase + (2 * gi + 3) * CB, CB)], didx1,
                       sem1)
      return carry

    lax.fori_loop(0, ROWS_PER_TILE // 2, grp, 0)
    pltpu.make_async_copy(d.at[pl.ds(0, CB)], didx0, sem0).wait()
    pltpu.make_async_copy(d.at[pl.ds(0, CB)], didx1, sem1).wait()
    plsc.subcore_barrier()
    for j in range(NRI):
      pltpu.sync_copy(ridxf.at[pl.ds((sub * NRI + j) * CB, CB)], rowidx)
      pltpu.sync_copy(acc.at[rowidx], zbuf)
      base = core * (3 * NP) + c * NP + sub * NPT + j * CB
      pltpu.sync_copy(zbuf, out.at[pl.ds(base, CB)])
    plsc.subcore_barrier()


_deg_call = pl.kernel(
    _deg_body,
    out_type=jax.ShapeDtypeStruct((2 * 3 * NP, DH), jnp.float32),
    mesh=_mesh,
    scratch_types=[
        pltpu.VMEM((CB,), jnp.int32),
        pltpu.VMEM((CB,), jnp.int32),
        pltpu.VMEM((CB, DH), jnp.float32),
        pltpu.VMEM((CB, DH), jnp.float32),
        pltpu.VMEM((CB,), jnp.int32),
        pltpu.SemaphoreType.DMA,
        pltpu.SemaphoreType.DMA,
        pltpu.VMEM_SHARED((NP, DH), jnp.float32),
    ],
)


def _scat_body(g, sflat, dflat, zer128, ridxf, out, sidx0, sidx1, didx0,
               didx1, rowidx, rows0, rows1, semi0, semi1, semg0, semg1, acc):
  core = lax.axis_index("c")
  sub = lax.axis_index("s")
  wid = core * NS + sub
  pltpu.sync_copy(zer128, rows0)
  for j in range(NRI):
    pltpu.sync_copy(ridxf.at[pl.ds((sub * NRI + j) * CB, CB)], rowidx)
    pltpu.sync_copy(rows0, acc.at[rowidx])
  plsc.subcore_barrier()

  ebase = wid * ROWS_PER_TILE * CB
  pltpu.async_copy(sflat.at[pl.ds(ebase, CB)], sidx0, semi0)
  pltpu.async_copy(dflat.at[pl.ds(ebase, CB)], didx0, semi0)
  pltpu.async_copy(sflat.at[pl.ds(ebase + CB, CB)], sidx1, semi1)
  pltpu.async_copy(dflat.at[pl.ds(ebase + CB, CB)], didx1, semi1)

  def grp(gi, carry):
    pltpu.make_async_copy(sflat.at[pl.ds(0, CB)], sidx0, semi0).wait()
    pltpu.make_async_copy(dflat.at[pl.ds(0, CB)], didx0, semi0).wait()
    gcp0 = pltpu.async_copy(g.at[sidx0], rows0, semg0)
    pltpu.make_async_copy(sflat.at[pl.ds(0, CB)], sidx1, semi1).wait()
    pltpu.make_async_copy(dflat.at[pl.ds(0, CB)], didx1, semi1).wait()
    gcp1 = pltpu.async_copy(g.at[sidx1], rows1, semg1)
    gcp0.wait()
    pltpu.sync_copy(rows0, acc.at[didx0], add=True)
    pltpu.async_copy(sflat.at[pl.ds(ebase + (2 * gi + 2) * CB, CB)], sidx0,
                     semi0)
    pltpu.async_copy(dflat.at[pl.ds(ebase + (2 * gi + 2) * CB, CB)], didx0,
                     semi0)
    gcp1.wait()
    pltpu.sync_copy(rows1, acc.at[didx1], add=True)
    pltpu.async_copy(sflat.at[pl.ds(ebase + (2 * gi + 3) * CB, CB)], sidx1,
                     semi1)
    pltpu.async_copy(dflat.at[pl.ds(ebase + (2 * gi + 3) * CB, CB)], didx1,
                     semi1)
    return carry

  lax.fori_loop(0, ROWS_PER_TILE // 2, grp, 0)
  pltpu.make_async_copy(sflat.at[pl.ds(0, CB)], sidx0, semi0).wait()
  pltpu.make_async_copy(dflat.at[pl.ds(0, CB)], didx0, semi0).wait()
  pltpu.make_async_copy(sflat.at[pl.ds(0, CB)], sidx1, semi1).wait()
  pltpu.make_async_copy(dflat.at[pl.ds(0, CB)], didx1, semi1).wait()
  plsc.subcore_barrier()
  for j in range(NRI):
    pltpu.sync_copy(ridxf.at[pl.ds((sub * NRI + j) * CB, CB)], rowidx)
    pltpu.sync_copy(acc.at[rowidx], rows0)
    base = core * NP + sub * NPT + j * CB
    pltpu.sync_copy(rows0, out.at[pl.ds(base, CB)])


_scat_call = pl.kernel(
    _scat_body,
    out_type=jax.ShapeDtypeStruct((2 * NP, DH), jnp.float32),
    mesh=_mesh,
    scratch_types=[
        pltpu.VMEM((CB,), jnp.int32),
        pltpu.VMEM((CB,), jnp.int32),
        pltpu.VMEM((CB,), jnp.int32),
        pltpu.VMEM((CB,), jnp.int32),
        pltpu.VMEM((CB,), jnp.int32),
        pltpu.VMEM((CB, DH), jnp.float32),
        pltpu.VMEM((CB, DH), jnp.float32),
        pltpu.SemaphoreType.DMA,
        pltpu.SemaphoreType.DMA,
        pltpu.SemaphoreType.DMA,
        pltpu.SemaphoreType.DMA,
        pltpu.VMEM_SHARED((NP, DH), jnp.float32),
    ],
)

R = 1000
_GRID = (N // R,)


def _mask_from(attp_ref):
  att = attp_ref[...]
  m = jnp.exp(att - jnp.max(att, axis=-1, keepdims=True))
  return m / jnp.sum(m, axis=-1, keepdims=True)


def _dinv_from(deg_ref):
  deg = deg_ref[...]
  return lax.rsqrt(deg[0, :, :, 0] + deg[1, :, :, 0] + 1.0)


def _tcA_body(x_ref, wcat_ref, w1_ref, b1_ref, w2_ref, b2_ref, attp_ref,
              deg_ref, g0, g1, g2, g3, hd):
  xb = x_ref[...]
  mask = _mask_from(attp_ref)
  dinv = _dinv_from(deg_ref)
  h_all = jnp.dot(xb, wcat_ref[...], preferred_element_type=jnp.float32)
  for c, (ref, s) in enumerate(zip((g0, g1, g2, g3), (2, 0, 1, 2))):
    ref[...] = dinv[s][:, None] * h_all[:, c * DH:(c + 1) * DH]
  h1 = jnp.maximum(
      jnp.dot(xb, w1_ref[...], preferred_element_type=jnp.float32)
      + b1_ref[...], 0.0)
  h2 = jnp.maximum(
      jnp.dot(h1, w2_ref[...], preferred_element_type=jnp.float32)
      + b2_ref[...], 0.0)
  hd[...] = h2 * mask[:, 3:4]


def _full(shape):
  return pl.BlockSpec(shape, lambda i: tuple(0 for _ in shape))


def _rows(width):
  return pl.BlockSpec((R, width), lambda i: (i, 0))


_deg_spec = pl.BlockSpec((2, 3, R, DH), lambda i: (0, 0, i, 0))
_part_spec = pl.BlockSpec((2, R, DH), lambda i: (0, i, 0))

_tcA_call = pl.pallas_call(
    _tcA_body,
    grid=_GRID,
    in_specs=[
        _rows(DH),
        _full((DH, 4 * DH)),
        _full((DH, DH)),
        _full((1, DH)),
        _full((DH, DH)),
        _full((1, DH)),
        _full((1, DH)),
        _deg_spec,
    ],
    out_specs=[_rows(DH)] * 5,
    out_shape=[jax.ShapeDtypeStruct((NP, DH), jnp.float32)] * 4
    + [jax.ShapeDtypeStruct((N, DH), jnp.float32)],
)


def _tcC_body(deg_ref, hd_ref, g0, g1, g2, g3, s0, s1, s2, s3, bconv_ref,
              bc0_ref, wc1_ref, attp_ref, base_ref, gc1_ref):
  mask = _mask_from(attp_ref)
  dinv = _dinv_from(deg_ref)
  emb = jnp.zeros_like(hd_ref[...])
  for i, (gref, sref) in enumerate(zip((g1, g2, g3), (s1, s2, s3))):
    sv = sref[0] + sref[1] + gref[...]
    out = dinv[i][:, None] * sv + bconv_ref[i:i + 1, :]
    emb = emb + jnp.maximum(out, 0.0) * mask[:, i:i + 1]
  sv = s0[0] + s0[1] + g0[...]
  y = jnp.maximum(dinv[2][:, None] * sv + bc0_ref[...], 0.0)
  gc1_ref[...] = dinv[2][:, None] * jnp.dot(
      y, wc1_ref[...], preferred_element_type=jnp.float32)
  base_ref[...] = emb + hd_ref[...]


_tcC_call = pl.pallas_call(
    _tcC_body,
    grid=_GRID,
    in_specs=[
        _deg_spec,
        _rows(DH),
        _rows(DH), _rows(DH), _rows(DH), _rows(DH),
        _part_spec, _part_spec, _part_spec, _part_spec,
        _full((3, DH)),
        _full((1, DH)),
        _full((DH, DH)),
        _full((1, DH)),
    ],
    out_specs=[_rows(DH), _rows(DH)],
    out_shape=[
        jax.ShapeDtypeStruct((N, DH), jnp.float32),
        jax.ShapeDtypeStruct((NP, DH), jnp.float32),
    ],
)


def _tcD_body(deg_ref, base_ref, gc1_ref, sc1_ref, bc1_ref, wf_ref, bf_ref,
              attp_ref, z_ref):
  mask = _mask_from(attp_ref)
  dinv = _dinv_from(deg_ref)
  sv = sc1_ref[0] + sc1_ref[1] + gc1_ref[...]
  extra = jnp.maximum(dinv[2][:, None] * sv + bc1_ref[...], 0.0) * mask[:, 4:5]
  final = base_ref[...] + extra
  u = jnp.dot(final, wf_ref[...], preferred_element_type=jnp.float32) \
      + bf_ref[...]
  mx = jnp.max(u, axis=-1, keepdims=True)
  z_ref[...] = u - (mx + jnp.log(jnp.sum(jnp.exp(u - mx), axis=-1,
                                         keepdims=True)))


_tcD_call = pl.pallas_call(
    _tcD_body,
    grid=_GRID,
    in_specs=[
        _deg_spec,
        _rows(DH),
        _rows(DH),
        _part_spec,
        _full((1, DH)),
        _full((DH, DOUT)),
        _full((1, DOUT)),
        _full((1, DH)),
    ],
    out_specs=_rows(DOUT),
    out_shape=jax.ShapeDtypeStruct((N, DOUT), jnp.float32),
)


@jax.jit
def kernel(x, edge_indexes, edge_attrs, W1, b1, W2, b2, Wc0, bc0, Wc1, bc1,
           Wconv, bconv, Wf, bf, att):
  del edge_attrs
  ei = edge_indexes.astype(jnp.int32)
  pad = jnp.full((EPP - E,), NP - 1, jnp.int32)
  src = [jnp.concatenate([ei[c, 0], pad]) for c in range(3)]
  dst = [jnp.concatenate([ei[c, 1], pad]) for c in range(3)]

  ridxf = jnp.arange(NP, dtype=jnp.int32)
  one128 = jnp.ones((CB, DH), jnp.float32)
  zer128 = jnp.zeros((CB, DH), jnp.float32)

  deg4 = _deg_call(dst[0], dst[1], dst[2], one128, zer128, ridxf)
  deg4 = deg4.reshape(2, 3, NP, DH)

  attp = jnp.full((1, DH), -1e30, jnp.float32).at[0, :att.shape[0]].set(att)
  wcat = jnp.concatenate([Wc0, Wconv[0], Wconv[1], Wconv[2]], axis=0).T
  g0, g1, g2, g3, hd = _tcA_call(x, wcat, W1.T, b1[None], W2.T, b2[None],
                                 attp, deg4)

  sp = []
  for gc, s in zip((g0, g1, g2, g3), (2, 0, 1, 2)):
    sp.append(_scat_call(gc, src[s], dst[s], zer128, ridxf).reshape(2, NP, DH))

  base, gc1 = _tcC_call(deg4, hd, g0, g1, g2, g3, sp[0], sp[1], sp[2], sp[3],
                        bconv, bc0[None], Wc1.T, attp)
  spc1 = _scat_call(gc1, src[2], dst[2], zer128, ridxf).reshape(2, NP, DH)
  z = _tcD_call(deg4, base, gc1, spc1, bc1[None], Wf.T, bf[None], attp)
  return z
